# baseline (speedup 1.0000x reference)
import jax
import jax.numpy as jnp
from jax.experimental import pallas as pl

_PLACEHOLDER = 42
_BN = 512


def _matvec_body(s_ref, w_ref, b_ref, o_ref):
    o_ref[...] = (
        jnp.dot(s_ref[...], w_ref[...], preferred_element_type=jnp.float32)
        + b_ref[...]
    )


def _select_body(x_ref, tok_ref, rep_ref, eye_ref, o_ref):
    i = pl.program_id(0)
    tok = tok_ref[:, pl.ds(i * _BN, _BN)]
    maskf = (tok == _PLACEHOLDER).astype(jnp.float32)
    mask_col = jax.lax.dot_general(
        eye_ref[...], maskf,
        dimension_numbers=(((1,), (1,)), ((), ())),
        preferred_element_type=jnp.float32,
    )
    o_ref[...] = jnp.where(mask_col > 0.5, rep_ref[...], x_ref[...])


def kernel(tokenized_text, embedded_text, special_embeddings, W, b):
    B, N, D = embedded_text.shape
    R = B * N
    x = embedded_text.reshape(R, D)
    tok = tokenized_text.reshape(1, R).astype(jnp.int32)
    s = special_embeddings.reshape(1, D)
    bias = b.reshape(1, D)

    rep = pl.pallas_call(
        _matvec_body,
        out_shape=jax.ShapeDtypeStruct((1, D), jnp.float32),
    )(s, W, bias)

    eye = jnp.eye(_BN, dtype=jnp.float32)
    out = pl.pallas_call(
        _select_body,
        grid=(R // _BN,),
        in_specs=[
            pl.BlockSpec((_BN, D), lambda i: (i, 0)),
            pl.BlockSpec((1, R), lambda i: (0, 0)),
            pl.BlockSpec((1, D), lambda i: (0, 0)),
            pl.BlockSpec((_BN, _BN), lambda i: (0, 0)),
        ],
        out_specs=pl.BlockSpec((_BN, D), lambda i: (i, 0)),
        out_shape=jax.ShapeDtypeStruct((R, D), jnp.float32),
    )(x, tok, rep, eye)
    return out.reshape(B, N, D)


# speedup vs baseline: 1.0471x; 1.0471x over previous
import jax
import jax.numpy as jnp
from jax.experimental import pallas as pl
from jax.experimental.pallas import tpu as pltpu

_PLACEHOLDER = 42
_BN = 512


def _body(tok_ref, x_ref, s_ref, w_ref, b_ref, o_ref, rep_ref, eye_ref):
    i = pl.program_id(0)

    @pl.when(i == 0)
    def _init():
        rep_ref[...] = (
            jnp.dot(s_ref[...], w_ref[...], preferred_element_type=jnp.float32)
            + b_ref[...]
        )
        rows = jax.lax.broadcasted_iota(jnp.int32, (_BN, _BN), 0)
        cols = jax.lax.broadcasted_iota(jnp.int32, (_BN, _BN), 1)
        eye_ref[...] = (rows == cols).astype(jnp.float32)

    N = tok_ref.shape[1]
    r = i * _BN
    tok = tok_ref[pl.ds(r // N, 1), pl.ds(r % N, _BN)]
    maskf = (tok == _PLACEHOLDER).astype(jnp.float32)
    mask_col = jax.lax.dot_general(
        eye_ref[...], maskf,
        dimension_numbers=(((1,), (1,)), ((), ())),
        preferred_element_type=jnp.float32,
    )
    o_ref[...] = jnp.where(mask_col > 0.5, rep_ref[...], x_ref[...])


def kernel(tokenized_text, embedded_text, special_embeddings, W, b):
    B, N, D = embedded_text.shape
    R = B * N
    x = embedded_text.reshape(R, D)
    tok = tokenized_text.astype(jnp.int32)
    s = special_embeddings.reshape(1, D)
    bias = b.reshape(1, D)

    out = pl.pallas_call(
        _body,
        grid=(R // _BN,),
        in_specs=[
            pl.BlockSpec((B, N), lambda i: (0, 0)),
            pl.BlockSpec((_BN, D), lambda i: (i, 0)),
            pl.BlockSpec((1, D), lambda i: (0, 0)),
            pl.BlockSpec((D, D), lambda i: (0, 0)),
            pl.BlockSpec((1, D), lambda i: (0, 0)),
        ],
        out_specs=pl.BlockSpec((_BN, D), lambda i: (i, 0)),
        out_shape=jax.ShapeDtypeStruct((R, D), jnp.float32),
        scratch_shapes=[
            pltpu.VMEM((1, D), jnp.float32),
            pltpu.VMEM((_BN, _BN), jnp.float32),
        ],
    )(tok, x, s, W, bias)
    return out.reshape(B, N, D)


# speedup vs baseline: 1.1378x; 1.0866x over previous
import jax
import jax.numpy as jnp
from jax.experimental import pallas as pl
from jax.experimental.pallas import tpu as pltpu

_PLACEHOLDER = 42
_BN = 1024


def _body(tok_ref, x_ref, s_ref, w_ref, b_ref, o_ref, rep_ref, eye_ref):
    i = pl.program_id(0)

    @pl.when(i == 0)
    def _init():
        rep_ref[...] = (
            jnp.dot(s_ref[...], w_ref[...], preferred_element_type=jnp.float32)
            + b_ref[...]
        )
        rows = jax.lax.broadcasted_iota(jnp.int32, (_BN, _BN), 0)
        cols = jax.lax.broadcasted_iota(jnp.int32, (_BN, _BN), 1)
        eye_ref[...] = (rows == cols).astype(jnp.float32)

    N = tok_ref.shape[1]
    r = i * _BN
    tok = tok_ref[pl.ds(r // N, 1), pl.ds(r % N, _BN)]
    maskf = (tok == _PLACEHOLDER).astype(jnp.float32)
    mask_col = jax.lax.dot_general(
        eye_ref[...], maskf,
        dimension_numbers=(((1,), (1,)), ((), ())),
        preferred_element_type=jnp.float32,
    )
    o_ref[...] = jnp.where(mask_col > 0.5, rep_ref[...], x_ref[...])


def kernel(tokenized_text, embedded_text, special_embeddings, W, b):
    B, N, D = embedded_text.shape
    R = B * N
    x = embedded_text.reshape(R, D)
    tok = tokenized_text.astype(jnp.int32)
    s = special_embeddings.reshape(1, D)
    bias = b.reshape(1, D)

    out = pl.pallas_call(
        _body,
        grid=(R // _BN,),
        in_specs=[
            pl.BlockSpec((B, N), lambda i: (0, 0)),
            pl.BlockSpec((_BN, D), lambda i: (i, 0)),
            pl.BlockSpec((1, D), lambda i: (0, 0)),
            pl.BlockSpec((D, D), lambda i: (0, 0)),
            pl.BlockSpec((1, D), lambda i: (0, 0)),
        ],
        out_specs=pl.BlockSpec((_BN, D), lambda i: (i, 0)),
        out_shape=jax.ShapeDtypeStruct((R, D), jnp.float32),
        scratch_shapes=[
            pltpu.VMEM((1, D), jnp.float32),
            pltpu.VMEM((_BN, _BN), jnp.float32),
        ],
    )(tok, x, s, W, bias)
    return out.reshape(B, N, D)


# speedup vs baseline: 1.1768x; 1.0343x over previous
import jax
import jax.numpy as jnp
from jax.experimental import pallas as pl
from jax.experimental.pallas import tpu as pltpu

_PLACEHOLDER = 42
_BN = 2048
_C = 128


def _body(tok_ref, x_ref, s_ref, w_ref, b_ref, o_ref, rep_ref, eye_ref):
    i = pl.program_id(0)

    @pl.when(i == 0)
    def _init():
        rep_ref[...] = (
            jnp.dot(s_ref[...], w_ref[...], preferred_element_type=jnp.float32)
            + b_ref[...]
        )
        rows = jax.lax.broadcasted_iota(jnp.int32, (_C, _C), 0)
        cols = jax.lax.broadcasted_iota(jnp.int32, (_C, _C), 1)
        eye_ref[...] = (rows == cols).astype(jnp.float32)

    N = tok_ref.shape[1]
    r = i * _BN
    rep = rep_ref[...]
    tok_row = tok_ref[pl.ds(r // N, 1), pl.ds(r % N, _BN)]
    for c in range(_BN // _C):
        tok = jax.lax.slice(tok_row, (0, c * _C), (1, (c + 1) * _C))
        maskf = (tok == _PLACEHOLDER).astype(jnp.float32)
        mask_col = jax.lax.dot_general(
            eye_ref[...], maskf,
            dimension_numbers=(((1,), (1,)), ((), ())),
            preferred_element_type=jnp.float32,
        )
        sl = pl.ds(c * _C, _C)
        o_ref[sl, :] = jnp.where(mask_col > 0.5, rep, x_ref[sl, :])


def kernel(tokenized_text, embedded_text, special_embeddings, W, b):
    B, N, D = embedded_text.shape
    R = B * N
    x = embedded_text.reshape(R, D)
    tok = tokenized_text.astype(jnp.int32)
    s = special_embeddings.reshape(1, D)
    bias = b.reshape(1, D)

    out = pl.pallas_call(
        _body,
        grid=(R // _BN,),
        in_specs=[
            pl.BlockSpec((B, N), lambda i: (0, 0)),
            pl.BlockSpec((_BN, D), lambda i: (i, 0)),
            pl.BlockSpec((1, D), lambda i: (0, 0)),
            pl.BlockSpec((D, D), lambda i: (0, 0)),
            pl.BlockSpec((1, D), lambda i: (0, 0)),
        ],
        out_specs=pl.BlockSpec((_BN, D), lambda i: (i, 0)),
        out_shape=jax.ShapeDtypeStruct((R, D), jnp.float32),
        scratch_shapes=[
            pltpu.VMEM((1, D), jnp.float32),
            pltpu.VMEM((_C, _C), jnp.float32),
        ],
    )(tok, x, s, W, bias)
    return out.reshape(B, N, D)
